# Initial kernel scaffold; baseline (speedup 1.0000x reference)
import jax
import jax.numpy as jnp
from jax.experimental import pallas as pl


def kernel(x, weight):
    raise NotImplementedError("write your pallas kernel here")


# speedup vs baseline: 1.3976x; 1.3976x over previous
import functools

import jax
import jax.numpy as jnp
from jax import lax
from jax.experimental import pallas as pl
from jax.experimental.pallas import tpu as pltpu
from jax.experimental.pallas import tpu_sc as plsc

_SCALE = 10.0
_D = 32
_LANES = 16


@functools.partial(jax.jit, static_argnames=("b_per_w", "chunk", "n_workers"))
def _scaled_embedding(x_flat, weight, *, b_per_w, chunk, n_workers):
    n_chunks = b_per_w // chunk
    num_b = x_flat.shape[0]
    mesh = plsc.VectorSubcoreMesh(core_axis_name="c", subcore_axis_name="s")
    n_cores = mesh.num_cores

    @functools.partial(
        pl.kernel,
        out_type=jax.ShapeDtypeStruct((num_b, _D), jnp.float32),
        mesh=mesh,
        scratch_types=[
            pltpu.VMEM((b_per_w,), jnp.int32),
            pltpu.VMEM((chunk, _D), jnp.float32),
            pltpu.VMEM((chunk, _D), jnp.float32),
            pltpu.SemaphoreType.DMA,
            pltpu.SemaphoreType.DMA,
        ],
        compiler_params=pltpu.CompilerParams(use_tc_tiling_on_sc=False),
    )
    def body(x_hbm, w_hbm, out_hbm, idx_v, buf0, buf1, sem0, sem1):
        wid = lax.axis_index("s") * n_cores + lax.axis_index("c")
        base = wid * b_per_w
        pltpu.sync_copy(x_hbm.at[pl.ds(base, b_per_w)], idx_v)

        bufs = (buf0, buf1)
        sems = (sem0, sem1)

        def start_gather(g):
            p = g % 2
            return pltpu.async_copy(
                w_hbm.at[idx_v.at[pl.ds(g * chunk, chunk)]], bufs[p], sems[p]
            )

        descs = [None] * n_chunks
        descs[0] = start_gather(0)
        if n_chunks > 1:
            descs[1] = start_gather(1)

        for g in range(n_chunks):
            p = g % 2
            buf = bufs[p]
            descs[g].wait()

            @plsc.parallel_loop(0, chunk, unroll=8)
            def _(i, _buf=buf):
                _buf[i, pl.ds(0, _LANES)] = _buf[i, pl.ds(0, _LANES)] * _SCALE
                _buf[i, pl.ds(_LANES, _LANES)] = (
                    _buf[i, pl.ds(_LANES, _LANES)] * _SCALE
                )

            pltpu.sync_copy(buf, out_hbm.at[pl.ds(base + g * chunk, chunk)])
            if g + 2 < n_chunks:
                descs[g + 2] = start_gather(g + 2)

    return body(x_flat, weight)


def kernel(x, weight):
    b0, b1 = x.shape
    num_b = b0 * b1
    x_flat = x.reshape(num_b)
    if x_flat.dtype != jnp.int32:
        x_flat = x_flat.astype(jnp.int32)
    n_workers = 32
    b_per_w = num_b // n_workers
    out = _scaled_embedding(
        x_flat, weight, b_per_w=b_per_w, chunk=1664, n_workers=n_workers
    )
    return out.reshape(b0, b1, _D)
